# baseline (speedup 1.0000x reference)
import functools

import jax
import jax.numpy as jnp
from jax import lax
from jax.experimental import pallas as pl
from jax.experimental.pallas import tpu as pltpu
from jax.experimental.pallas import tpu_sc as plsc

_EMB = 64
_BATCH = 4096
_HIST = 50
_NTOT = _BATCH * _HIST
_VOCAB2 = 500000

_info = plsc.get_sparse_core_info()
_NC, _NS = _info.num_cores, _info.num_subcores
_NW = _NC * _NS
_B_PER_W = _NTOT // _NW
_CHUNK = 128
_NCHUNK = _B_PER_W // _CHUNK
_NGRP = _CHUNK // 16

_mesh = plsc.VectorSubcoreMesh(core_axis_name="c", subcore_axis_name="s")


@functools.partial(
    pl.kernel,
    mesh=_mesh,
    out_type=jax.ShapeDtypeStruct((_NTOT,), jnp.int32),
    scratch_types=[
        pltpu.VMEM((8, 128), jnp.int32),
    ],
)
def _detile_idx(idxt_hbm, out_hbm, tile_v):
    wid = lax.axis_index("s") * _NC + lax.axis_index("c")
    col = wid * 128
    for a in range(7):
        rows = 8 if a < 6 else 2
        pltpu.sync_copy(
            idxt_hbm.at[pl.ds(a * 8, rows), pl.ds(col, 128)],
            tile_v.at[pl.ds(0, rows)],
        )
        for s in range(rows):
            pltpu.sync_copy(
                tile_v.at[s],
                out_hbm.at[pl.ds((a * 8 + s) * _BATCH + col, 128)],
            )


@functools.partial(
    pl.kernel,
    mesh=_mesh,
    out_type=jax.ShapeDtypeStruct((_HIST, _EMB, _BATCH), jnp.float32),
    scratch_types=[
        pltpu.VMEM((_B_PER_W,), jnp.int32),
        pltpu.VMEM((_B_PER_W,), jnp.int32),
        pltpu.VMEM((2, _CHUNK, 128), jnp.float32),
        pltpu.VMEM((2, _EMB, _CHUNK), jnp.float32),
        pltpu.SemaphoreType.DMA,
        pltpu.SemaphoreType.DMA,
    ],
    compiler_params=pltpu.CompilerParams(needs_layout_passes=False),
)
def _emb_lookup(idx_hbm, table_hbm, out_hbm, gidx_v, poff_v, rows_v, slab_v,
                gsem0, gsem1):
    wid = lax.axis_index("s") * _NC + lax.axis_index("c")
    base = wid * _B_PER_W
    pltpu.sync_copy(idx_hbm.at[pl.ds(base, _B_PER_W)], gidx_v)

    def prep(i, _):
        raw = gidx_v[pl.ds(i * 16, 16)]
        poff_v[pl.ds(i * 16, 16)] = (raw & 1) << 6
        gidx_v[pl.ds(i * 16, 16)] = lax.shift_right_logical(raw, 1)
        return _

    lax.fori_loop(0, _B_PER_W // 16, prep, 0)

    iota16 = lax.iota(jnp.int32, 16)

    def extract_and_store(c, buf):
        for g in range(_NGRP):
            row_ids = iota16 + (g * 16)
            p16 = poff_v[pl.ds(c * _CHUNK + g * 16, 16)]
            for e in range(_EMB):
                vals = plsc.load_gather(
                    rows_v.at[buf], [row_ids, p16 + e]
                )
                slab_v[buf, e, pl.ds(g * 16, 16)] = vals
        grow = base + c * _CHUNK
        h = lax.shift_right_logical(grow, 12)
        b0 = pl.multiple_of(grow & (_BATCH - 1), _CHUNK)
        pltpu.sync_copy(
            slab_v.at[buf],
            out_hbm.at[h, :, pl.ds(b0, _CHUNK)],
        )

    pltpu.async_copy(
        table_hbm.at[gidx_v.at[pl.ds(0, _CHUNK)]], rows_v.at[0], gsem0
    )

    def body(p, _):
        c0 = 2 * p
        pltpu.async_copy(
            table_hbm.at[gidx_v.at[pl.ds((c0 + 1) * _CHUNK, _CHUNK)]],
            rows_v.at[1],
            gsem1,
        )
        pltpu.make_async_copy(
            table_hbm.at[gidx_v.at[pl.ds(0, _CHUNK)]], rows_v.at[0], gsem0
        ).wait()
        extract_and_store(c0, 0)
        nxt = lax.min(c0 + 2, _NCHUNK - 2)
        pltpu.async_copy(
            table_hbm.at[gidx_v.at[pl.ds(nxt * _CHUNK, _CHUNK)]],
            rows_v.at[0],
            gsem0,
        )
        pltpu.make_async_copy(
            table_hbm.at[gidx_v.at[pl.ds(0, _CHUNK)]], rows_v.at[1], gsem1
        ).wait()
        extract_and_store(c0 + 1, 1)
        return _

    lax.fori_loop(0, _NCHUNK // 2, body, 0)
    pltpu.make_async_copy(
        table_hbm.at[gidx_v.at[pl.ds(0, _CHUNK)]], rows_v.at[0], gsem0
    ).wait()


def kernel(input, weight):
    idxt = jnp.transpose(input.astype(jnp.int32))
    flat = _detile_idx(idxt)
    w128 = jnp.reshape(weight, (_VOCAB2, 128))
    out3 = _emb_lookup(flat, w128)
    return jnp.transpose(out3, (2, 0, 1))
